# Initial kernel scaffold; baseline (speedup 1.0000x reference)
import jax
import jax.numpy as jnp
from jax.experimental import pallas as pl


def kernel(x, table, W, b):
    raise NotImplementedError("write your pallas kernel here")


# speedup vs baseline: 8.6143x; 8.6143x over previous
import functools

import jax
import jax.numpy as jnp
from jax import lax
from jax.experimental import pallas as pl
from jax.experimental.pallas import tpu as pltpu
from jax.experimental.pallas import tpu_sc as plsc

VOCAB = 1_000_000
EMBED = 32
OUT = 32
PACK = 4
TBLK = 2000
NW = 32
CH = 128
L = 16


def _transform_body(t_ref, w_ref, b_ref, o_ref):
    o_ref[...] = (
        jnp.dot(t_ref[...], w_ref[...], preferred_element_type=jnp.float32)
        + b_ref[...]
    )


def _transform(table, W, b):
    packed = table.reshape(VOCAB // PACK, EMBED * PACK)
    bd = jnp.kron(jnp.eye(PACK, dtype=jnp.float32), W.T)
    bt = jnp.tile(b, PACK).reshape(1, EMBED * PACK)
    grid = (VOCAB // PACK) // TBLK
    return pl.pallas_call(
        _transform_body,
        grid=(grid,),
        in_specs=[
            pl.BlockSpec((TBLK, EMBED * PACK), lambda i: (i, 0)),
            pl.BlockSpec((EMBED * PACK, EMBED * PACK), lambda i: (0, 0)),
            pl.BlockSpec((1, EMBED * PACK), lambda i: (0, 0)),
        ],
        out_specs=pl.BlockSpec((TBLK, EMBED * PACK), lambda i: (i, 0)),
        out_shape=jax.ShapeDtypeStruct((VOCAB // PACK, EMBED * PACK), jnp.float32),
    )(packed, bd, bt)


def _gather(tbl4, idx):
    B = idx.shape[0]
    b_per_w = B // NW
    n_steps = b_per_w // CH
    mesh = plsc.VectorSubcoreMesh(core_axis_name="c", subcore_axis_name="s")

    @functools.partial(
        pl.kernel,
        mesh=mesh,
        out_type=jax.ShapeDtypeStruct((B * EMBED,), jnp.float32),
        compiler_params=pltpu.CompilerParams(needs_layout_passes=False),
        scratch_types=[
            pltpu.VMEM((CH,), jnp.int32),
            pltpu.VMEM((CH,), jnp.int32),
            pltpu.VMEM((CH, PACK * EMBED), jnp.float32),
            pltpu.VMEM((CH * EMBED,), jnp.float32),
            pltpu.SemaphoreType.DMA,
        ],
    )
    def k(tbl_hbm, idx_hbm, out_hbm, idx_v, q_v, wide_v, out_v, sem):
        wid = lax.axis_index("s") * 2 + lax.axis_index("c")
        base = wid * b_per_w

        def body(step, carry):
            off = base + step * CH
            pltpu.sync_copy(idx_hbm.at[pl.ds(off, CH)], idx_v)
            for g in range(CH // L):
                q_v[pl.ds(g * L, L)] = lax.shift_right_logical(
                    idx_v[pl.ds(g * L, L)], 2
                )
            pltpu.async_copy(tbl_hbm.at[q_v], wide_v, sem).wait()
            for g in range(CH // L):
                m = lax.bitwise_and(idx_v[pl.ds(g * L, L)], 3)
                colbase = lax.shift_left(m, 5)
                rows16 = g * L + lax.iota(jnp.int32, L)
                outbase = EMBED * rows16
                for j in range(EMBED):
                    vals = plsc.load_gather(wide_v, [rows16, colbase + j])
                    plsc.store_scatter(out_v, [outbase + j], vals)
            pltpu.sync_copy(out_v, out_hbm.at[pl.ds(off * EMBED, CH * EMBED)])
            return carry

        lax.fori_loop(0, n_steps, body, 0)

    return k(tbl4, idx)


def kernel(x, table, W, b):
    T4 = _transform(table, W, b)
    flat = x.reshape(-1).astype(jnp.int32)
    out = _gather(T4, flat)
    return out.reshape(x.shape + (OUT,))


# speedup vs baseline: 10.3976x; 1.2070x over previous
import functools

import jax
import jax.numpy as jnp
from jax import lax
from jax.experimental import pallas as pl
from jax.experimental.pallas import tpu as pltpu
from jax.experimental.pallas import tpu_sc as plsc

VOCAB = 1_000_000
EMBED = 32
OUT = 32
PACK = 4
WIDE = PACK * EMBED
TBLK = 10000
NW = 32
CH = 128
L = 16


def _transform_body(t_ref, w_ref, b_ref, o_ref):
    o_ref[...] = (
        jnp.dot(t_ref[...], w_ref[...], preferred_element_type=jnp.float32)
        + b_ref[...]
    )


def _transform(table, W, b):
    packed = table.reshape(VOCAB // PACK, WIDE)
    bd = jnp.kron(jnp.eye(PACK, dtype=jnp.float32), W.T)
    bt = jnp.tile(b, PACK).reshape(1, WIDE)
    grid = (VOCAB // PACK) // TBLK
    return pl.pallas_call(
        _transform_body,
        grid=(grid,),
        in_specs=[
            pl.BlockSpec((TBLK, WIDE), lambda i: (i, 0)),
            pl.BlockSpec((WIDE, WIDE), lambda i: (0, 0)),
            pl.BlockSpec((1, WIDE), lambda i: (0, 0)),
        ],
        out_specs=pl.BlockSpec((TBLK, WIDE), lambda i: (i, 0)),
        out_shape=jax.ShapeDtypeStruct((VOCAB // PACK, WIDE), jnp.float32),
    )(packed, bd, bt)


def _gather(tbl4, idx):
    B = idx.shape[0]
    b_per_w = B // NW
    n_stages = b_per_w // CH
    n_pairs = n_stages // 2
    mesh = plsc.VectorSubcoreMesh(core_axis_name="c", subcore_axis_name="s")

    @functools.partial(
        pl.kernel,
        mesh=mesh,
        out_type=jax.ShapeDtypeStruct((B * EMBED,), jnp.float32),
        compiler_params=pltpu.CompilerParams(needs_layout_passes=False),
        scratch_types=[
            pltpu.VMEM((b_per_w,), jnp.int32),
            [pltpu.VMEM((CH,), jnp.int32)] * 2,
            [pltpu.VMEM((CH, WIDE), jnp.float32)] * 2,
            [pltpu.VMEM((CH * EMBED,), jnp.float32)] * 2,
            [pltpu.SemaphoreType.DMA] * 2,
            [pltpu.SemaphoreType.DMA] * 2,
        ],
    )
    def k(tbl_hbm, idx_hbm, out_hbm, idx_all, q_v, wide_v, out_v, sg, sw):
        wid = lax.axis_index("s") * 2 + lax.axis_index("c")
        base = wid * b_per_w
        pltpu.sync_copy(idx_hbm.at[pl.ds(base, b_per_w)], idx_all)

        def compute_q(s, buf):
            for g in range(CH // L):
                q_v[buf][pl.ds(g * L, L)] = lax.shift_right_logical(
                    idx_all[pl.ds(s * CH + g * L, L)], 2
                )

        def fire_gather(buf):
            pltpu.async_copy(tbl_hbm.at[q_v[buf]], wide_v[buf], sg[buf])

        def wait_gather(buf):
            pltpu.make_async_copy(tbl_hbm.at[q_v[buf]], wide_v[buf], sg[buf]).wait()

        def extract(s, buf):
            for g in range(CH // L):
                m = lax.bitwise_and(idx_all[pl.ds(s * CH + g * L, L)], 3)
                colbase = lax.shift_left(m, 5)
                rows16 = g * L + lax.iota(jnp.int32, L)
                outbase = EMBED * rows16
                for j in range(EMBED):
                    vals = plsc.load_gather(wide_v[buf], [rows16, colbase + j])
                    plsc.store_scatter(out_v[buf], [outbase + j], vals)

        def fire_writeback(s, buf):
            off = (base + s * CH) * EMBED
            pltpu.async_copy(out_v[buf], out_hbm.at[pl.ds(off, CH * EMBED)], sw[buf])

        def drain_writeback(buf):
            pltpu.make_async_copy(
                out_hbm.at[pl.ds(base * EMBED, CH * EMBED)], out_v[buf], sw[buf]
            ).wait()

        compute_q(0, 0)
        fire_gather(0)

        def body(p, carry):
            s0 = 2 * p
            s1 = s0 + 1
            compute_q(s1, 1)
            fire_gather(1)
            wait_gather(0)

            @pl.when(p != 0)
            def _():
                drain_writeback(0)

            extract(s0, 0)
            fire_writeback(s0, 0)

            @pl.when(p != n_pairs - 1)
            def _():
                compute_q(s0 + 2, 0)
                fire_gather(0)

            wait_gather(1)

            @pl.when(p != 0)
            def _():
                drain_writeback(1)

            extract(s1, 1)
            fire_writeback(s1, 1)
            return carry

        lax.fori_loop(0, n_pairs, body, 0)
        drain_writeback(0)
        drain_writeback(1)

    return k(tbl4, idx)


def kernel(x, table, W, b):
    T4 = _transform(table, W, b)
    flat = x.reshape(-1).astype(jnp.int32)
    out = _gather(T4, flat)
    return out.reshape(x.shape + (OUT,))
